# baseline (speedup 1.0000x reference)
import functools

import jax
import jax.numpy as jnp
from jax import lax
from jax.experimental import pallas as pl
from jax.experimental.pallas import tpu as pltpu
from jax.experimental.pallas import tpu_sc as plsc

VOCAB_SIZE = 1000000
TIME_DIM = 32
BATCH = 4096
SEQ_LEN = 50

_B_TOTAL = BATCH * SEQ_LEN
_NUM_WORKERS = 32
_B_PER_W = _B_TOTAL // _NUM_WORKERS
_CHUNK = 1600
_NSTEPS = _B_PER_W // _CHUNK
_NBUF = 2


@functools.partial(
    pl.kernel,
    mesh=plsc.VectorSubcoreMesh(core_axis_name="c", subcore_axis_name="s"),
    out_type=jax.ShapeDtypeStruct((_B_TOTAL, TIME_DIM), jnp.float32),
    scratch_types=[
        pltpu.VMEM((_B_PER_W,), jnp.int32),
        pltpu.VMEM((_NBUF, _CHUNK, TIME_DIM), jnp.float32),
        pltpu.SemaphoreType.DMA,
    ],
    compiler_params=pltpu.CompilerParams(use_tc_tiling_on_sc=False),
)
def _gather_kernel(table_hbm, idx_hbm, out_hbm, idx_v, rows_v, gsem):
    wid = lax.axis_index("s") * 2 + lax.axis_index("c")
    base = wid * _B_PER_W
    pltpu.sync_copy(idx_hbm.at[pl.ds(base, _B_PER_W)], idx_v)

    pltpu.async_copy(
        table_hbm.at[idx_v.at[pl.ds(0, _CHUNK)]], rows_v.at[0], gsem
    )
    for s in range(_NSTEPS):
        if s + 1 < _NSTEPS:
            pltpu.async_copy(
                table_hbm.at[idx_v.at[pl.ds((s + 1) * _CHUNK, _CHUNK)]],
                rows_v.at[(s + 1) % _NBUF],
                gsem,
            )
        pltpu.make_async_copy(
            table_hbm.at[idx_v.at[pl.ds(s * _CHUNK, _CHUNK)]],
            rows_v.at[s % _NBUF],
            gsem,
        ).wait()
        pltpu.sync_copy(
            rows_v.at[s % _NBUF],
            out_hbm.at[pl.ds(base + s * _CHUNK, _CHUNK)],
        )


def kernel(token_ids, time_embeddings):
    idx = token_ids.reshape(-1).astype(jnp.int32)
    out = _gather_kernel(time_embeddings, idx)
    return out.reshape(token_ids.shape + (TIME_DIM,))


# speedup vs baseline: 1.1464x; 1.1464x over previous
import functools

import jax
import jax.numpy as jnp
from jax import lax
from jax.experimental import pallas as pl
from jax.experimental.pallas import tpu as pltpu
from jax.experimental.pallas import tpu_sc as plsc

VOCAB_SIZE = 1000000
TIME_DIM = 32
BATCH = 4096
SEQ_LEN = 50

_NW = 32
_BB = BATCH // _NW
_GROUPED_ROWS = VOCAB_SIZE // 4


@functools.partial(
    pl.kernel,
    mesh=plsc.VectorSubcoreMesh(core_axis_name="c", subcore_axis_name="s"),
    out_type=jax.ShapeDtypeStruct((SEQ_LEN, TIME_DIM, BATCH), jnp.float32),
    scratch_types=[
        pltpu.VMEM((SEQ_LEN, _BB), jnp.int32),
        pltpu.VMEM((2, _BB), jnp.int32),
        pltpu.VMEM((2, _BB), jnp.int32),
        pltpu.VMEM((2, _BB, 128), jnp.float32),
        pltpu.VMEM((2, TIME_DIM, _BB), jnp.float32),
        pltpu.SemaphoreType.DMA,
        pltpu.SemaphoreType.DMA,
    ],
    compiler_params=pltpu.CompilerParams(
        use_tc_tiling_on_sc=True, needs_layout_passes=False),
)
def _sc_gather(tab_rm, t_ids, out_t, idx_v, ridx_v, coff_v, rows_v, ostage_v,
               sem0, sem1):
    w = lax.axis_index("s") * 2 + lax.axis_index("c")
    sems = (sem0, sem1)
    pltpu.sync_copy(t_ids.at[:, pl.ds(w * _BB, _BB)], idx_v)

    def prep_and_fire(s, p):
        for k in range(_BB // 16):
            v = idx_v[s, pl.ds(k * 16, 16)]
            ridx_v[p, pl.ds(k * 16, 16)] = lax.shift_right_logical(v, 2)
            coff_v[p, pl.ds(k * 16, 16)] = lax.shift_left(
                lax.bitwise_and(v, 3), 5)
        pltpu.async_copy(tab_rm.at[ridx_v.at[p]], rows_v.at[p], sems[p])

    def drain_extract_write(s, p):
        pltpu.make_async_copy(tab_rm.at[ridx_v.at[p]], rows_v.at[p],
                              sems[p]).wait()
        rows2d = rows_v.at[p]
        for k in range(_BB // 16):
            tvec = lax.iota(jnp.int32, 16) + (k * 16)
            cvec = coff_v[p, pl.ds(k * 16, 16)]
            for f in range(TIME_DIM):
                ostage_v[p, f, pl.ds(k * 16, 16)] = plsc.load_gather(
                    rows2d, [tvec, cvec + f])
        pltpu.sync_copy(ostage_v.at[p],
                        out_t.at[s, :, pl.ds(w * _BB, _BB)])

    prep_and_fire(0, 0)

    def body(i, carry):
        for j in range(2):
            s = 2 * i + j
            @pl.when(s + 1 < SEQ_LEN)
            def _():
                prep_and_fire(s + 1, (j + 1) % 2)
            drain_extract_write(s, j)
        return carry

    lax.fori_loop(0, SEQ_LEN // 2, body, 0)


def kernel(token_ids, time_embeddings):
    tab_rm = time_embeddings.reshape(_GROUPED_ROWS, 128)
    t_ids = token_ids.T.astype(jnp.int32)
    out_t = _sc_gather(tab_rm, t_ids)
    return out_t.transpose(2, 0, 1)
